# Initial kernel scaffold; baseline (speedup 1.0000x reference)
import jax
import jax.numpy as jnp
from jax.experimental import pallas as pl


def kernel(gsp, gsp_time_utc_fourier, gsp_solar_azimuth, gsp_solar_elevation, gsp_y_osgb_fourier, gsp_x_osgb_fourier, gsp_time_utc_fourier_t0, gsp_id, gsp_id_embedding_weight, include_history, do_reshape_time_as_batch):
    raise NotImplementedError("write your pallas kernel here")


# speedup vs baseline: 1.9526x; 1.9526x over previous
import functools

import jax
import jax.numpy as jnp
from jax import lax
from jax.experimental import pallas as pl
from jax.experimental.pallas import tpu as pltpu
from jax.experimental.pallas import tpu_sc as plsc

NC, NS, L = 2, 16, 16
CB = 8


@functools.lru_cache(maxsize=None)
def _build(B, T, F, V, D):
    NW = NC * NS
    assert B % (NW * CB) == 0 and F == L and D % L == 0
    C = 1 + 4 * F + 2 + D
    M_AZ = 1 + 4 * F
    M_EL = M_AZ + 1
    BPW = B // NW
    NCHUNK = BPW // CB
    ROWS = CB * T

    mesh = plsc.VectorSubcoreMesh(
        core_axis_name="c", subcore_axis_name="s",
        num_cores=NC, num_subcores=NS)

    @functools.partial(
        pl.kernel,
        out_type=jax.ShapeDtypeStruct((B * T * C,), jnp.float32),
        mesh=mesh,
        compiler_params=pltpu.CompilerParams(
            use_tc_tiling_on_sc=False, needs_layout_passes=False),
        scratch_types=[
            pltpu.VMEM((CB,), jnp.int32),
            pltpu.VMEM((CB, D), jnp.float32),
            pltpu.VMEM((CB * F,), jnp.float32),
            pltpu.VMEM((CB * F,), jnp.float32),
            pltpu.VMEM((CB * F,), jnp.float32),
            pltpu.VMEM((CB * T * F,), jnp.float32),
            pltpu.VMEM((CB * T,), jnp.float32),
            pltpu.VMEM((CB * T,), jnp.float32),
            pltpu.VMEM((ROWS * C,), jnp.float32),
            pltpu.VMEM((L,), jnp.float32),
            pltpu.SemaphoreType.DMA,
        ],
    )
    def k(time_h, az_h, el_h, y_h, x_h, t0_h, ids_h, tab_h, mark_h, out_h,
          idx_v, emb_v, y_v, x_v, t0_v, time_v, az_v, el_v, outb, mark_v,
          gsem):
        wid = lax.axis_index("s") * NC + lax.axis_index("c")
        iota = lax.iota(jnp.int32, L)
        pltpu.sync_copy(mark_h, mark_v)
        mk = mark_v[...]
        pre = ([iota + 1, iota + (1 + F), iota + (1 + 2 * F),
                iota + (1 + 3 * F)]
               + [iota + (M_EL + 1 + L * kk) for kk in range(D // L)])

        def chunk(cc, carry):
            b0 = wid * BPW + cc * CB
            pltpu.sync_copy(ids_h.at[pl.ds(b0, CB)], idx_v)
            g = pltpu.async_copy(tab_h.at[idx_v], emb_v, gsem)
            pltpu.sync_copy(time_h.at[pl.ds(b0 * T * F, CB * T * F)], time_v)
            pltpu.sync_copy(az_h.at[pl.ds(b0 * T, CB * T)], az_v)
            pltpu.sync_copy(el_h.at[pl.ds(b0 * T, CB * T)], el_v)
            pltpu.sync_copy(y_h.at[pl.ds(b0 * F, CB * F)], y_v)
            pltpu.sync_copy(x_h.at[pl.ds(b0 * F, CB * F)], x_v)
            pltpu.sync_copy(t0_h.at[pl.ds(b0 * F, CB * F)], t0_v)
            g.wait()

            def per_ex(i, carry2):
                yv = y_v[pl.ds(i * F, F)]
                xv = x_v[pl.ds(i * F, F)]
                t0v = t0_v[pl.ds(i * F, F)]
                evs = [emb_v[i, pl.ds(kk * L, L)] for kk in range(D // L)]

                def per_t(t, carry3):
                    r = i * T + t
                    tv = time_v[pl.ds(r * F, F)]
                    rb = jnp.full((L,), r * C, jnp.int32)
                    plsc.store_scatter(outb, [rb + pre[0]], yv)
                    plsc.store_scatter(outb, [rb + pre[1]], xv)
                    plsc.store_scatter(outb, [rb + pre[2]], tv)
                    plsc.store_scatter(outb, [rb + pre[3]], t0v)
                    for kk in range(D // L):
                        plsc.store_scatter(outb, [rb + pre[4 + kk]], evs[kk])
                    return carry3

                lax.fori_loop(0, T, per_t, 0)
                return carry2

            lax.fori_loop(0, CB, per_ex, 0)

            def cols(j, carry2):
                addr = (j * L + iota) * C
                plsc.store_scatter(outb, [addr], mk)
                plsc.store_scatter(outb, [addr + M_AZ],
                                   az_v[pl.ds(j * L, L)])
                plsc.store_scatter(outb, [addr + M_EL],
                                   el_v[pl.ds(j * L, L)])
                return carry2

            lax.fori_loop(0, ROWS // L, cols, 0)

            pltpu.sync_copy(outb, out_h.at[pl.ds(b0 * T * C, ROWS * C)])
            return carry

        lax.fori_loop(0, NCHUNK, chunk, 0)

    return k


def kernel(gsp, gsp_time_utc_fourier, gsp_solar_azimuth, gsp_solar_elevation,
           gsp_y_osgb_fourier, gsp_x_osgb_fourier, gsp_time_utc_fourier_t0,
           gsp_id, gsp_id_embedding_weight, include_history=0,
           do_reshape_time_as_batch=1):
    B, T, F = gsp_time_utc_fourier.shape
    V, D = gsp_id_embedding_weight.shape
    C = 1 + 4 * F + 2 + D
    marker = jnp.full((L,), 1.0, jnp.float32) * (
        jnp.asarray(do_reshape_time_as_batch, jnp.float32)
        + jnp.asarray(include_history, jnp.float32))
    out_flat = _build(B, T, F, V, D)(
        gsp_time_utc_fourier.reshape(-1),
        gsp_solar_azimuth.reshape(-1),
        gsp_solar_elevation.reshape(-1),
        gsp_y_osgb_fourier.reshape(-1),
        gsp_x_osgb_fourier.reshape(-1),
        gsp_time_utc_fourier_t0.reshape(-1),
        gsp_id.reshape(-1).astype(jnp.int32),
        gsp_id_embedding_weight,
        marker,
    )
    return out_flat.reshape(B * T, 1, C)


# speedup vs baseline: 2.2863x; 1.1709x over previous
import functools

import jax
import jax.numpy as jnp
from jax import lax
from jax.experimental import pallas as pl
from jax.experimental.pallas import tpu as pltpu
from jax.experimental.pallas import tpu_sc as plsc

NC, NS, L = 2, 16, 16
CB = 4
UNROLL = 4


@functools.lru_cache(maxsize=None)
def _build(B, T, F, V, D):
    NW = NC * NS
    assert B % (NW * 2 * CB) == 0 and F == L and D % L == 0
    C = 1 + 4 * F + 2 + D
    M_AZ = 1 + 4 * F
    M_EL = M_AZ + 1
    BPW = B // NW
    NCH = BPW // CB
    ROWS = CB * T

    mesh = plsc.VectorSubcoreMesh(
        core_axis_name="c", subcore_axis_name="s",
        num_cores=NC, num_subcores=NS)

    @functools.partial(
        pl.kernel,
        out_type=jax.ShapeDtypeStruct((B * T * C,), jnp.float32),
        mesh=mesh,
        compiler_params=pltpu.CompilerParams(
            use_tc_tiling_on_sc=False, needs_layout_passes=False),
        scratch_types=[
            pltpu.VMEM((BPW,), jnp.int32),
            pltpu.VMEM((BPW, D), jnp.float32),
            pltpu.VMEM((BPW * F,), jnp.float32),
            pltpu.VMEM((BPW * F,), jnp.float32),
            pltpu.VMEM((BPW * F,), jnp.float32),
            pltpu.VMEM((L,), jnp.float32),
            [pltpu.VMEM((ROWS * F,), jnp.float32) for _ in range(2)],
            [pltpu.VMEM((ROWS,), jnp.float32) for _ in range(2)],
            [pltpu.VMEM((ROWS,), jnp.float32) for _ in range(2)],
            [pltpu.VMEM((ROWS * C,), jnp.float32) for _ in range(2)],
            [pltpu.SemaphoreType.DMA for _ in range(2)],
            [pltpu.SemaphoreType.DMA for _ in range(2)],
        ],
    )
    def k(time_h, az_h, el_h, y_h, x_h, t0_h, ids_h, tab_h, mark_h, out_h,
          ids_v, emb_v, y_v, x_v, t0_v, mark_v, time_v, az_v, el_v, outb,
          isem, osem):
        wid = lax.axis_index("s") * NC + lax.axis_index("c")
        w0 = wid * BPW
        iota = lax.iota(jnp.int32, L)
        pltpu.sync_copy(ids_h.at[pl.ds(w0, BPW)], ids_v)
        g = pltpu.async_copy(tab_h.at[ids_v], emb_v, isem[0])
        pltpu.sync_copy(y_h.at[pl.ds(w0 * F, BPW * F)], y_v)
        pltpu.sync_copy(x_h.at[pl.ds(w0 * F, BPW * F)], x_v)
        pltpu.sync_copy(t0_h.at[pl.ds(w0 * F, BPW * F)], t0_v)
        pltpu.sync_copy(mark_h, mark_v)
        mk = mark_v[...]
        g.wait()
        pre = ([iota + 1, iota + (1 + F), iota + (1 + 2 * F),
                iota + (1 + 3 * F)]
               + [iota + (M_EL + 1 + L * kk) for kk in range(D // L)])

        def fire_inputs(ch, p):
            r0 = (w0 + ch * CB) * T
            pltpu.async_copy(time_h.at[pl.ds(r0 * F, ROWS * F)],
                             time_v[p], isem[p])
            pltpu.async_copy(az_h.at[pl.ds(r0, ROWS)], az_v[p], isem[p])
            pltpu.async_copy(el_h.at[pl.ds(r0, ROWS)], el_v[p], isem[p])

        def drain_inputs(p):
            pltpu.make_async_copy(time_h.at[pl.ds(0, ROWS * F)],
                                  time_v[p], isem[p]).wait()
            pltpu.make_async_copy(az_h.at[pl.ds(0, ROWS)],
                                  az_v[p], isem[p]).wait()
            pltpu.make_async_copy(el_h.at[pl.ds(0, ROWS)],
                                  el_v[p], isem[p]).wait()

        def drain_out(p):
            pltpu.make_async_copy(out_h.at[pl.ds(0, ROWS * C)],
                                  outb[p], osem[p]).wait()

        def do_phase(cc, p):
            ch = 2 * cc + p
            fire_inputs(jnp.minimum(ch + 1, NCH - 1), 1 - p)
            drain_inputs(p)

            @pl.when(cc > 0)
            def _():
                drain_out(p)

            for i in range(CB):
                e = ch * CB + i

                def scat(vec, pidx, rb):
                    plsc.store_scatter(outb[p], [rb + pre[pidx]], vec)

                yv = y_v[pl.ds(e * F, F)]
                xv = x_v[pl.ds(e * F, F)]
                t0v = t0_v[pl.ds(e * F, F)]
                evs = [emb_v[e, pl.ds(kk * L, L)] for kk in range(D // L)]

                @plsc.parallel_loop(0, T, 1, unroll=UNROLL)
                def rowbody(t):
                    r = i * T + t
                    tv = time_v[p][pl.ds(r * F, F)]
                    rb = jnp.full((L,), r * C, jnp.int32)
                    scat(yv, 0, rb)
                    scat(xv, 1, rb)
                    scat(tv, 2, rb)
                    scat(t0v, 3, rb)
                    for kk in range(D // L):
                        scat(evs[kk], 4 + kk, rb)

            @plsc.parallel_loop(0, ROWS // L, 1, unroll=UNROLL)
            def colbody(j):
                addr = (j * L + iota) * C
                plsc.store_scatter(outb[p], [addr], mk)
                plsc.store_scatter(outb[p], [addr + M_AZ],
                                   az_v[p][pl.ds(j * L, L)])
                plsc.store_scatter(outb[p], [addr + M_EL],
                                   el_v[p][pl.ds(j * L, L)])

            r0 = (w0 + ch * CB) * T
            pltpu.async_copy(outb[p], out_h.at[pl.ds(r0 * C, ROWS * C)],
                             osem[p])

        fire_inputs(0, 0)

        def body(cc, carry):
            do_phase(cc, 0)
            do_phase(cc, 1)
            return carry

        lax.fori_loop(0, NCH // 2, body, 0)
        drain_out(0)
        drain_out(1)
        drain_inputs(0)

    return k


def kernel(gsp, gsp_time_utc_fourier, gsp_solar_azimuth, gsp_solar_elevation,
           gsp_y_osgb_fourier, gsp_x_osgb_fourier, gsp_time_utc_fourier_t0,
           gsp_id, gsp_id_embedding_weight, include_history=0,
           do_reshape_time_as_batch=1):
    B, T, F = gsp_time_utc_fourier.shape
    V, D = gsp_id_embedding_weight.shape
    C = 1 + 4 * F + 2 + D
    marker = jnp.full((L,), 1.0, jnp.float32) * (
        jnp.asarray(do_reshape_time_as_batch, jnp.float32)
        + jnp.asarray(include_history, jnp.float32))
    out_flat = _build(B, T, F, V, D)(
        gsp_time_utc_fourier.reshape(-1),
        gsp_solar_azimuth.reshape(-1),
        gsp_solar_elevation.reshape(-1),
        gsp_y_osgb_fourier.reshape(-1),
        gsp_x_osgb_fourier.reshape(-1),
        gsp_time_utc_fourier_t0.reshape(-1),
        gsp_id.reshape(-1).astype(jnp.int32),
        gsp_id_embedding_weight,
        marker,
    )
    return out_flat.reshape(B * T, 1, C)


# speedup vs baseline: 4.1883x; 1.8319x over previous
import functools

import jax
import jax.numpy as jnp
from jax import lax
from jax.experimental import pallas as pl
from jax.experimental.pallas import tpu as pltpu
from jax.experimental.pallas import tpu_sc as plsc

NC, NS, L = 2, 16, 16
CB = 4
HB = 64
UNROLL = 4


@functools.lru_cache(maxsize=None)
def _build(B, T, F, V, D):
    NW = NC * NS
    assert B % (NW * 2 * CB) == 0 and F == L and D % L == 0
    C = 1 + 4 * F + 2 + D
    M_AZ = 1 + 4 * F
    M_EL = M_AZ + 1
    BPW = B // NW
    NCH = BPW // CB
    ROWS = CB * T
    CPH = HB // CB

    mesh = plsc.VectorSubcoreMesh(
        core_axis_name="c", subcore_axis_name="s",
        num_cores=NC, num_subcores=NS)

    @functools.partial(
        pl.kernel,
        out_type=jax.ShapeDtypeStruct((B * T, C), jnp.float32),
        mesh=mesh,
        compiler_params=pltpu.CompilerParams(
            use_tc_tiling_on_sc=True, needs_layout_passes=False),
        scratch_types=[
            pltpu.VMEM((HB,), jnp.int32),
            pltpu.VMEM((HB, D), jnp.float32),
            pltpu.VMEM((HB * F,), jnp.float32),
            pltpu.VMEM((HB * F,), jnp.float32),
            pltpu.VMEM((HB * F,), jnp.float32),
            pltpu.VMEM((L,), jnp.float32),
            [pltpu.VMEM((ROWS * F,), jnp.float32) for _ in range(2)],
            [pltpu.VMEM((ROWS + L,), jnp.float32) for _ in range(2)],
            [pltpu.VMEM((ROWS + L,), jnp.float32) for _ in range(2)],
            [pltpu.VMEM((ROWS, C), jnp.float32) for _ in range(2)],
            [pltpu.SemaphoreType.DMA for _ in range(2)],
            [pltpu.SemaphoreType.DMA for _ in range(2)],
        ],
    )
    def k(time_h, az_h, el_h, y_h, x_h, t0_h, ids_h, tab_h, mark_h, out_h,
          ids_v, emb_v, y_v, x_v, t0_v, mark_v, time_v, az_v, el_v, outb,
          isem, osem):
        wid = lax.axis_index("s") * NC + lax.axis_index("c")
        w0 = wid * BPW
        iota = lax.iota(jnp.int32, L)
        pltpu.sync_copy(mark_h, mark_v)
        mk = mark_v[...]
        pre = ([iota + 1, iota + (1 + F), iota + (1 + 2 * F),
                iota + (1 + 3 * F)]
               + [iota + (M_EL + 1 + L * kk) for kk in range(D // L)])

        def stage_half(h):
            h0 = w0 + h * HB
            pltpu.sync_copy(ids_h.at[pl.ds(h0, HB)], ids_v)
            g = pltpu.async_copy(tab_h.at[ids_v], emb_v, isem[0])
            pltpu.sync_copy(y_h.at[pl.ds(h0 * F, HB * F)], y_v)
            pltpu.sync_copy(x_h.at[pl.ds(h0 * F, HB * F)], x_v)
            pltpu.sync_copy(t0_h.at[pl.ds(h0 * F, HB * F)], t0_v)
            g.wait()

        def fire_inputs(ch, p):
            r0 = (w0 + ch * CB) * T
            pltpu.async_copy(time_h.at[pl.ds(r0 * F, ROWS * F)],
                             time_v[p], isem[p])
            pltpu.async_copy(az_h.at[pl.ds(r0, ROWS)],
                             az_v[p].at[pl.ds(0, ROWS)], isem[p])
            pltpu.async_copy(el_h.at[pl.ds(r0, ROWS)],
                             el_v[p].at[pl.ds(0, ROWS)], isem[p])

        def drain_inputs(p):
            pltpu.make_async_copy(time_h.at[pl.ds(0, ROWS * F)],
                                  time_v[p], isem[p]).wait()
            pltpu.make_async_copy(az_h.at[pl.ds(0, ROWS)],
                                  az_v[p].at[pl.ds(0, ROWS)], isem[p]).wait()
            pltpu.make_async_copy(el_h.at[pl.ds(0, ROWS)],
                                  el_v[p].at[pl.ds(0, ROWS)], isem[p]).wait()

        def drain_out(p):
            pltpu.make_async_copy(out_h.at[pl.ds(0, ROWS), :],
                                  outb[p], osem[p]).wait()

        def do_phase(h, cc, p, first):
            ch = 2 * cc + p
            fire_inputs(jnp.minimum(h * CPH + ch + 1, NCH - 1), 1 - p)
            drain_inputs(p)

            if first:
                @pl.when(cc > 0)
                def _():
                    drain_out(p)
            else:
                drain_out(p)

            for i in range(CB):
                e = ch * CB + i

                def scat(vec, pidx, rv):
                    plsc.store_scatter(outb[p], [rv, pre[pidx]], vec)

                yv = y_v[pl.ds(e * F, F)]
                xv = x_v[pl.ds(e * F, F)]
                t0v = t0_v[pl.ds(e * F, F)]
                evs = [emb_v[e, pl.ds(kk * L, L)] for kk in range(D // L)]

                @plsc.parallel_loop(0, T, 1, unroll=UNROLL)
                def rowbody(t):
                    r = i * T + t
                    tv = time_v[p][pl.ds(r * F, F)]
                    rv = jnp.full((L,), r, jnp.int32)
                    scat(yv, 0, rv)
                    scat(xv, 1, rv)
                    scat(tv, 2, rv)
                    scat(t0v, 3, rv)
                    for kk in range(D // L):
                        scat(evs[kk], 4 + kk, rv)

            @plsc.parallel_loop(0, ROWS // L, 1, unroll=UNROLL)
            def colbody(j):
                rows = j * L + iota
                plsc.store_scatter(outb[p],
                                   [rows, jnp.full((L,), 0, jnp.int32)], mk)
                plsc.store_scatter(outb[p],
                                   [rows, jnp.full((L,), M_AZ, jnp.int32)],
                                   az_v[p][pl.ds(j * L, L)])
                plsc.store_scatter(outb[p],
                                   [rows, jnp.full((L,), M_EL, jnp.int32)],
                                   el_v[p][pl.ds(j * L, L)])

            if ROWS % L:
                tb = (ROWS // L) * L
                tmask = iota < (ROWS - tb)
                trows = tb + iota
                plsc.store_scatter(outb[p],
                                   [trows, jnp.full((L,), 0, jnp.int32)],
                                   mk, mask=tmask)
                plsc.store_scatter(outb[p],
                                   [trows, jnp.full((L,), M_AZ, jnp.int32)],
                                   az_v[p][pl.ds(tb, L)], mask=tmask)
                plsc.store_scatter(outb[p],
                                   [trows, jnp.full((L,), M_EL, jnp.int32)],
                                   el_v[p][pl.ds(tb, L)], mask=tmask)

            r0 = (w0 + (h * CPH + ch) * CB) * T
            pltpu.async_copy(outb[p], out_h.at[pl.ds(r0, ROWS), :], osem[p])

        fire_inputs(0, 0)
        for h in range(2):
            stage_half(h)

            def body(cc, carry, h=h, first=(h == 0)):
                do_phase(h, cc, 0, first)
                do_phase(h, cc, 1, first)
                return carry

            lax.fori_loop(0, CPH // 2, body, 0)
        drain_out(0)
        drain_out(1)
        drain_inputs(0)

    return k


def kernel(gsp, gsp_time_utc_fourier, gsp_solar_azimuth, gsp_solar_elevation,
           gsp_y_osgb_fourier, gsp_x_osgb_fourier, gsp_time_utc_fourier_t0,
           gsp_id, gsp_id_embedding_weight, include_history=0,
           do_reshape_time_as_batch=1):
    B, T, F = gsp_time_utc_fourier.shape
    V, D = gsp_id_embedding_weight.shape
    C = 1 + 4 * F + 2 + D
    marker = jnp.full((L,), 1.0, jnp.float32) * (
        jnp.asarray(do_reshape_time_as_batch, jnp.float32)
        + jnp.asarray(include_history, jnp.float32))
    out2d = _build(B, T, F, V, D)(
        gsp_time_utc_fourier.reshape(-1),
        gsp_solar_azimuth.reshape(-1),
        gsp_solar_elevation.reshape(-1),
        gsp_y_osgb_fourier.reshape(-1),
        gsp_x_osgb_fourier.reshape(-1),
        gsp_time_utc_fourier_t0.reshape(-1),
        gsp_id.reshape(-1).astype(jnp.int32),
        gsp_id_embedding_weight,
        marker,
    )
    return out2d.reshape(B * T, 1, C)
